# baseline (speedup 1.0000x reference)
import functools

import jax
import jax.numpy as jnp
from jax import lax
from jax.experimental import pallas as pl
from jax.experimental.pallas import tpu as pltpu
from jax.experimental.pallas import tpu_sc as plsc

_B = 16384
_D = 32
_NC = 2
_NS = 16
_NW = _NC * _NS
_BW = _B // _NW
_CH = 128
_NCH = _BW // _CH

_NU = 1000000
_NM = 100000
_GRID = 32
_UBLK = 32768
_ULAST = 30
_MBLK = 4096
_MLAST = 24


def _mv_body(ut_ref, mt_ref, w_ref, b_ref, su_ref, sm_ref):
    su_ref[...] = jnp.sum(ut_ref[...] * w_ref[:_D, 0:1], axis=0)
    sm_ref[...] = jnp.sum(mt_ref[...] * w_ref[_D:, 0:1], axis=0) + b_ref[0]


def _matvecs(ut, mt, w_col, bias_val):
    return pl.pallas_call(
        _mv_body,
        grid=(_GRID,),
        in_specs=[
            pl.BlockSpec((_D, _UBLK), lambda i: (0, jnp.minimum(i, _ULAST))),
            pl.BlockSpec((_D, _MBLK), lambda i: (0, jnp.minimum(i, _MLAST))),
            pl.BlockSpec((2 * _D, 1), lambda i: (0, 0)),
            pl.BlockSpec(memory_space=pltpu.SMEM),
        ],
        out_specs=[
            pl.BlockSpec((_UBLK,), lambda i: (jnp.minimum(i, _ULAST),)),
            pl.BlockSpec((_MBLK,), lambda i: (jnp.minimum(i, _MLAST),)),
        ],
        out_shape=[
            jax.ShapeDtypeStruct((_NU,), jnp.float32),
            jax.ShapeDtypeStruct((_NM,), jnp.float32),
        ],
    )(ut, mt, w_col, bias_val)


_sc_mesh = plsc.VectorSubcoreMesh(core_axis_name="c", subcore_axis_name="s")


@functools.partial(
    pl.kernel,
    mesh=_sc_mesh,
    compiler_params=pltpu.CompilerParams(needs_layout_passes=False),
    out_type=jax.ShapeDtypeStruct((_B,), jnp.float32),
    scratch_types=[
        pltpu.VMEM((_BW,), jnp.int32),
        pltpu.VMEM((_BW,), jnp.int32),
        pltpu.VMEM((_BW,), jnp.float32),
        pltpu.VMEM((_BW,), jnp.float32),
        pltpu.VMEM((_BW,), jnp.float32),
        pltpu.SemaphoreType.DMA,
    ],
)
def _gather_combine(su_hbm, sm_hbm, users_hbm, movies_hbm, out_hbm,
                    uidx, midx, su, sm, outv, sem):
    wid = lax.axis_index("s") * _NC + lax.axis_index("c")
    base = wid * _BW

    pltpu.sync_copy(users_hbm.at[pl.ds(base, _BW)], uidx)
    pltpu.sync_copy(movies_hbm.at[pl.ds(base, _BW)], midx)

    copies = []
    for c in range(_NCH):
        sl = pl.ds(c * _CH, _CH)
        copies.append(pltpu.async_copy(su_hbm.at[uidx.at[sl]], su.at[sl], sem))
        copies.append(pltpu.async_copy(sm_hbm.at[midx.at[sl]], sm.at[sl], sem))
    for cp in copies:
        cp.wait()

    for g in range(_BW // 16):
        sl = pl.ds(g * 16, 16)
        outv[sl] = su[sl] + sm[sl]

    pltpu.sync_copy(outv, out_hbm.at[pl.ds(base, _BW)])


def kernel(users, movies, user_table, movie_table, W, b):
    w_col = W.reshape(2 * _D, 1).astype(jnp.float32)
    scores_u, scores_m = _matvecs(user_table.T, movie_table.T, w_col,
                                  b.astype(jnp.float32).reshape(1))
    out = _gather_combine(scores_u, scores_m,
                          users.astype(jnp.int32), movies.astype(jnp.int32))
    return out.reshape(_B, 1)


# speedup vs baseline: 1.1411x; 1.1411x over previous
import functools

import jax
import jax.numpy as jnp
from jax import lax
from jax.experimental import pallas as pl
from jax.experimental.pallas import tpu as pltpu
from jax.experimental.pallas import tpu_sc as plsc

_B = 16384
_D = 32
_NC = 2
_NS = 16
_NW = _NC * _NS
_BW = _B // _NW
_CH = 128
_NCH = _BW // _CH

_NU = 1000000
_NM = 100000
_GRID = 16
_UBLK = 65536
_ULAST = 15
_MBLK = 8192
_MLAST = 12


_DOT_DN = (((0,), (0,)), ((), ()))


def _mv_body(ut_ref, mt_ref, w_ref, b_ref, su_ref, sm_ref):
    su = jax.lax.dot_general(w_ref[:_D, :], ut_ref[...], _DOT_DN,
                             preferred_element_type=jnp.float32)
    sm = jax.lax.dot_general(w_ref[_D:, :], mt_ref[...], _DOT_DN,
                             preferred_element_type=jnp.float32)
    su_ref[...] = su[0]
    sm_ref[...] = sm[0] + b_ref[0]


def _matvecs(ut, mt, w_col, bias_val):
    return pl.pallas_call(
        _mv_body,
        grid=(_GRID,),
        in_specs=[
            pl.BlockSpec((_D, _UBLK), lambda i: (0, jnp.minimum(i, _ULAST))),
            pl.BlockSpec((_D, _MBLK), lambda i: (0, jnp.minimum(i, _MLAST))),
            pl.BlockSpec((2 * _D, 1), lambda i: (0, 0)),
            pl.BlockSpec(memory_space=pltpu.SMEM),
        ],
        out_specs=[
            pl.BlockSpec((_UBLK,), lambda i: (jnp.minimum(i, _ULAST),)),
            pl.BlockSpec((_MBLK,), lambda i: (jnp.minimum(i, _MLAST),)),
        ],
        out_shape=[
            jax.ShapeDtypeStruct((_NU,), jnp.float32),
            jax.ShapeDtypeStruct((_NM,), jnp.float32),
        ],
    )(ut, mt, w_col, bias_val)


_sc_mesh = plsc.VectorSubcoreMesh(core_axis_name="c", subcore_axis_name="s")


@functools.partial(
    pl.kernel,
    mesh=_sc_mesh,
    compiler_params=pltpu.CompilerParams(needs_layout_passes=False),
    out_type=jax.ShapeDtypeStruct((_B,), jnp.float32),
    scratch_types=[
        pltpu.VMEM((_BW,), jnp.int32),
        pltpu.VMEM((_BW,), jnp.int32),
        pltpu.VMEM((_BW,), jnp.float32),
        pltpu.VMEM((_BW,), jnp.float32),
        pltpu.VMEM((_BW,), jnp.float32),
        pltpu.SemaphoreType.DMA,
    ],
)
def _gather_combine(su_hbm, sm_hbm, users_hbm, movies_hbm, out_hbm,
                    uidx, midx, su, sm, outv, sem):
    wid = lax.axis_index("s") * _NC + lax.axis_index("c")
    base = wid * _BW

    pltpu.sync_copy(users_hbm.at[pl.ds(base, _BW)], uidx)
    pltpu.sync_copy(movies_hbm.at[pl.ds(base, _BW)], midx)

    copies = []
    for c in range(_NCH):
        sl = pl.ds(c * _CH, _CH)
        copies.append(pltpu.async_copy(su_hbm.at[uidx.at[sl]], su.at[sl], sem))
        copies.append(pltpu.async_copy(sm_hbm.at[midx.at[sl]], sm.at[sl], sem))
    for cp in copies:
        cp.wait()

    for g in range(_BW // 16):
        sl = pl.ds(g * 16, 16)
        outv[sl] = su[sl] + sm[sl]

    pltpu.sync_copy(outv, out_hbm.at[pl.ds(base, _BW)])


def kernel(users, movies, user_table, movie_table, W, b):
    w_col = W.reshape(2 * _D, 1).astype(jnp.float32)
    scores_u, scores_m = _matvecs(user_table.T, movie_table.T, w_col,
                                  b.astype(jnp.float32).reshape(1))
    out = _gather_combine(scores_u, scores_m,
                          users.astype(jnp.int32), movies.astype(jnp.int32))
    return out.reshape(_B, 1)
